# Initial kernel scaffold; baseline (speedup 1.0000x reference)
import jax
import jax.numpy as jnp
from jax.experimental import pallas as pl


def kernel(x, table):
    raise NotImplementedError("write your pallas kernel here")


# speedup vs baseline: 2.9640x; 2.9640x over previous
import functools

import jax
import jax.numpy as jnp
from jax import lax
from jax.experimental import pallas as pl
from jax.experimental.pallas import tpu as pltpu
from jax.experimental.pallas import tpu_sc as plsc

NUM_EMBEDDINGS = 100000
DIM = 128
BATCH = 4096
HIST = 50
TOTAL = BATCH * HIST

_info = plsc.get_sparse_core_info()
NC, NS = _info.num_cores, _info.num_subcores
NW = NC * NS
B_PER_W = TOTAL // NW
CHUNK = 128
N_CHUNKS = B_PER_W // CHUNK


@functools.partial(
    pl.kernel,
    mesh=plsc.VectorSubcoreMesh(core_axis_name="c", subcore_axis_name="s"),
    out_type=jax.ShapeDtypeStruct((TOTAL, DIM), jnp.float32),
    scratch_types=[
        pltpu.VMEM((N_CHUNKS, CHUNK), jnp.int32),
        pltpu.VMEM((CHUNK, DIM), jnp.float32),
        pltpu.SemaphoreType.DMA,
    ],
)
def _sc_gather(tab_hbm, idx_hbm, out_hbm, idx_v, rows_v, gsem):
    wid = lax.axis_index("s") * NC + lax.axis_index("c")
    base = wid * B_PER_W
    pltpu.sync_copy(idx_hbm.at[wid], idx_v)

    def body(c, carry):
        pltpu.async_copy(tab_hbm.at[idx_v.at[c]], rows_v, gsem).wait()
        pltpu.sync_copy(rows_v, out_hbm.at[pl.ds(base + c * CHUNK, CHUNK)])
        return carry

    lax.fori_loop(0, N_CHUNKS, body, 0)


def kernel(x, table):
    idx3d = x.reshape(NW, N_CHUNKS, CHUNK).astype(jnp.int32)
    out = _sc_gather(table, idx3d)
    return out.reshape(BATCH, HIST, DIM)


# speedup vs baseline: 3.3351x; 1.1252x over previous
import functools

import jax
import jax.numpy as jnp
from jax import lax
from jax.experimental import pallas as pl
from jax.experimental.pallas import tpu as pltpu
from jax.experimental.pallas import tpu_sc as plsc

NUM_EMBEDDINGS = 100000
DIM = 128
BATCH = 4096
HIST = 50
TOTAL = BATCH * HIST

_info = plsc.get_sparse_core_info()
NC, NS = _info.num_cores, _info.num_subcores
NW = NC * NS
B_PER_W = TOTAL // NW
CHUNK = 128
N_CHUNKS = B_PER_W // CHUNK


@functools.partial(
    pl.kernel,
    mesh=plsc.VectorSubcoreMesh(core_axis_name="c", subcore_axis_name="s"),
    out_type=jax.ShapeDtypeStruct((TOTAL, DIM), jnp.float32),
    scratch_types=[
        pltpu.VMEM((N_CHUNKS, CHUNK), jnp.int32),
        pltpu.VMEM((CHUNK, DIM), jnp.float32),
        pltpu.VMEM((CHUNK, DIM), jnp.float32),
        pltpu.SemaphoreType.DMA,
        pltpu.SemaphoreType.DMA,
        pltpu.SemaphoreType.DMA,
        pltpu.SemaphoreType.DMA,
    ],
)
def _sc_gather(tab_hbm, idx_hbm, out_hbm, idx_v, rows0, rows1, g0, g1, w0, w1):
    wid = lax.axis_index("s") * NC + lax.axis_index("c")
    base = wid * B_PER_W
    rows = (rows0, rows1)
    gsem = (g0, g1)
    wsem = (w0, w1)
    pltpu.sync_copy(idx_hbm.at[wid], idx_v)

    def start_g(c, b):
        pltpu.async_copy(tab_hbm.at[idx_v.at[c]], rows[b], gsem[b])

    def wait_g(c, b):
        pltpu.make_async_copy(tab_hbm.at[idx_v.at[c]], rows[b], gsem[b]).wait()

    def start_w(c, b):
        pltpu.async_copy(rows[b], out_hbm.at[pl.ds(base + c * CHUNK, CHUNK)], wsem[b])

    def wait_w(c, b):
        pltpu.make_async_copy(
            rows[b], out_hbm.at[pl.ds(base + c * CHUNK, CHUNK)], wsem[b]
        ).wait()

    start_g(0, 0)

    def body(o, carry):
        for k in (0, 1):
            c = 2 * o + k
            b = k
            ob = 1 - k

            @pl.when(c + 1 < N_CHUNKS)
            def _():
                @pl.when(c >= 1)
                def _():
                    wait_w(c - 1, ob)

                start_g(c + 1, ob)

            wait_g(c, b)
            start_w(c, b)
        return carry

    lax.fori_loop(0, N_CHUNKS // 2, body, 0)
    wait_w(N_CHUNKS - 2, 0)
    wait_w(N_CHUNKS - 1, 1)


def kernel(x, table):
    idx3d = x.reshape(NW, N_CHUNKS, CHUNK).astype(jnp.int32)
    out = _sc_gather(table, idx3d)
    return out.reshape(BATCH, HIST, DIM)


# speedup vs baseline: 3.3428x; 1.0023x over previous
import functools

import jax
import jax.numpy as jnp
from jax import lax
from jax.experimental import pallas as pl
from jax.experimental.pallas import tpu as pltpu
from jax.experimental.pallas import tpu_sc as plsc

NUM_EMBEDDINGS = 100000
DIM = 128
BATCH = 4096
HIST = 50
TOTAL = BATCH * HIST

_info = plsc.get_sparse_core_info()
NC, NS = _info.num_cores, _info.num_subcores
NW = NC * NS
B_PER_W = TOTAL // NW
CHUNK = 128
N_CHUNKS = B_PER_W // CHUNK


@functools.partial(
    pl.kernel,
    mesh=plsc.VectorSubcoreMesh(core_axis_name="c", subcore_axis_name="s"),
    out_type=jax.ShapeDtypeStruct((TOTAL, DIM), jnp.float32),
    scratch_types=[
        pltpu.VMEM((N_CHUNKS, CHUNK), jnp.int32),
        pltpu.VMEM((CHUNK, DIM), jnp.float32),
        pltpu.VMEM((CHUNK, DIM), jnp.float32),
        pltpu.VMEM((CHUNK, DIM), jnp.float32),
        pltpu.VMEM((CHUNK, DIM), jnp.float32),
        pltpu.SemaphoreType.DMA,
        pltpu.SemaphoreType.DMA,
        pltpu.SemaphoreType.DMA,
        pltpu.SemaphoreType.DMA,
        pltpu.SemaphoreType.DMA,
        pltpu.SemaphoreType.DMA,
        pltpu.SemaphoreType.DMA,
        pltpu.SemaphoreType.DMA,
    ],
)
def _sc_gather(
    tab_hbm, idx_hbm, out_hbm, idx_v,
    r0, r1, r2, r3, g0, g1, g2, g3, w0, w1, w2, w3,
):
    wid = lax.axis_index("s") * NC + lax.axis_index("c")
    base = wid * B_PER_W
    rows = (r0, r1, r2, r3)
    gsem = (g0, g1, g2, g3)
    wsem = (w0, w1, w2, w3)
    pltpu.sync_copy(idx_hbm.at[wid], idx_v)

    def start_g(c, b):
        pltpu.async_copy(tab_hbm.at[idx_v.at[c]], rows[b], gsem[b])

    def wait_g(c, b):
        pltpu.make_async_copy(tab_hbm.at[idx_v.at[c]], rows[b], gsem[b]).wait()

    def start_w(c, b):
        pltpu.async_copy(rows[b], out_hbm.at[pl.ds(base + c * CHUNK, CHUNK)], wsem[b])

    def wait_w(c, b):
        pltpu.make_async_copy(
            rows[b], out_hbm.at[pl.ds(base + c * CHUNK, CHUNK)], wsem[b]
        ).wait()

    start_g(0, 0)
    start_g(1, 1)

    def body(o, carry):
        for k in range(4):
            c = 4 * o + k
            b = k
            nb = (k + 2) % 4

            @pl.when(c >= 2)
            def _():
                wait_w(c - 2, nb)

            start_g(c + 2, nb)
            wait_g(c, b)
            start_w(c, b)
        return carry

    lax.fori_loop(0, (N_CHUNKS - 2) // 4, body, 0)
    for c in (N_CHUNKS - 2, N_CHUNKS - 1):
        b = c % 4
        wait_g(c, b)
        start_w(c, b)
    for c in range(N_CHUNKS - 4, N_CHUNKS):
        wait_w(c, c % 4)


def kernel(x, table):
    idx3d = x.reshape(NW, N_CHUNKS, CHUNK).astype(jnp.int32)
    out = _sc_gather(table, idx3d)
    return out.reshape(BATCH, HIST, DIM)


# speedup vs baseline: 5.9294x; 1.7738x over previous
import functools

import jax
import jax.numpy as jnp
from jax import lax
from jax.experimental import pallas as pl
from jax.experimental.pallas import tpu as pltpu
from jax.experimental.pallas import tpu_sc as plsc

NUM_EMBEDDINGS = 100000
DIM = 128
BATCH = 4096
HIST = 50

_info = plsc.get_sparse_core_info()
NC, NS = _info.num_cores, _info.num_subcores
NW = NC * NS
ROWS_PER_W = BATCH // NW
RPC = 4
N_CHUNKS = ROWS_PER_W // RPC


@functools.partial(
    pl.kernel,
    mesh=plsc.VectorSubcoreMesh(core_axis_name="c", subcore_axis_name="s"),
    out_type=jax.ShapeDtypeStruct((BATCH, HIST, DIM), jnp.float32),
    scratch_types=[
        pltpu.VMEM((ROWS_PER_W, HIST), jnp.int32),
        pltpu.VMEM((RPC, HIST, DIM), jnp.float32),
        pltpu.VMEM((RPC, HIST, DIM), jnp.float32),
        pltpu.SemaphoreType.DMA,
        pltpu.SemaphoreType.DMA,
        pltpu.SemaphoreType.DMA,
        pltpu.SemaphoreType.DMA,
    ],
)
def _sc_gather(tab_hbm, idx_hbm, out_hbm, idx_v, rows0, rows1, g0, g1, w0, w1):
    wid = lax.axis_index("s") * NC + lax.axis_index("c")
    base = wid * ROWS_PER_W
    rows = (rows0, rows1)
    gsem = (g0, g1)
    wsem = (w0, w1)
    pltpu.sync_copy(idx_hbm.at[pl.ds(base, ROWS_PER_W)], idx_v)

    def start_g(c, b):
        for r in range(RPC):
            pltpu.async_copy(
                tab_hbm.at[idx_v.at[c * RPC + r]], rows[b].at[r], gsem[b]
            )

    def wait_g(c, b):
        for r in range(RPC):
            pltpu.make_async_copy(
                tab_hbm.at[idx_v.at[c * RPC + r]], rows[b].at[r], gsem[b]
            ).wait()

    def start_w(c, b):
        pltpu.async_copy(rows[b], out_hbm.at[pl.ds(base + c * RPC, RPC)], wsem[b])

    def wait_w(c, b):
        pltpu.make_async_copy(
            rows[b], out_hbm.at[pl.ds(base + c * RPC, RPC)], wsem[b]
        ).wait()

    start_g(0, 0)

    def body(o, carry):
        for k in (0, 1):
            c = 2 * o + k
            b = k
            ob = 1 - k

            @pl.when(c + 1 < N_CHUNKS)
            def _():
                @pl.when(c >= 1)
                def _():
                    wait_w(c - 1, ob)

                start_g(c + 1, ob)

            wait_g(c, b)
            start_w(c, b)
        return carry

    lax.fori_loop(0, N_CHUNKS // 2, body, 0)
    wait_w(N_CHUNKS - 2, 0)
    wait_w(N_CHUNKS - 1, 1)


def kernel(x, table):
    return _sc_gather(table, x.astype(jnp.int32))


# speedup vs baseline: 10.4443x; 1.7614x over previous
import functools

import jax
import jax.numpy as jnp
from jax import lax
from jax.experimental import pallas as pl
from jax.experimental.pallas import tpu as pltpu
from jax.experimental.pallas import tpu_sc as plsc

NUM_EMBEDDINGS = 100000
DIM = 128
BATCH = 4096
HIST = 50

_info = plsc.get_sparse_core_info()
NC, NS = _info.num_cores, _info.num_subcores
NW = NC * NS
ROWS_PER_W = BATCH // NW


@functools.partial(
    pl.kernel,
    mesh=plsc.VectorSubcoreMesh(core_axis_name="c", subcore_axis_name="s"),
    out_type=jax.ShapeDtypeStruct((HIST, BATCH, DIM), jnp.float32),
    scratch_types=[
        pltpu.VMEM((HIST, ROWS_PER_W), jnp.int32),
        pltpu.VMEM((ROWS_PER_W, DIM), jnp.float32),
        pltpu.VMEM((ROWS_PER_W, DIM), jnp.float32),
        pltpu.SemaphoreType.DMA,
        pltpu.SemaphoreType.DMA,
        pltpu.SemaphoreType.DMA,
        pltpu.SemaphoreType.DMA,
    ],
)
def _sc_gather(tab_hbm, idx_hbm, out_hbm, idx_v, rows0, rows1, g0, g1, w0, w1):
    wid = lax.axis_index("s") * NC + lax.axis_index("c")
    base = wid * ROWS_PER_W
    rows = (rows0, rows1)
    gsem = (g0, g1)
    wsem = (w0, w1)
    pltpu.sync_copy(idx_hbm.at[:, pl.ds(base, ROWS_PER_W)], idx_v)

    def start_g(h, b):
        pltpu.async_copy(tab_hbm.at[idx_v.at[h]], rows[b], gsem[b])

    def wait_g(h, b):
        pltpu.make_async_copy(tab_hbm.at[idx_v.at[h]], rows[b], gsem[b]).wait()

    def start_w(h, b):
        pltpu.async_copy(rows[b], out_hbm.at[h, pl.ds(base, ROWS_PER_W)], wsem[b])

    def wait_w(h, b):
        pltpu.make_async_copy(
            rows[b], out_hbm.at[h, pl.ds(base, ROWS_PER_W)], wsem[b]
        ).wait()

    start_g(0, 0)

    def body(o, carry):
        for k in (0, 1):
            h = 2 * o + k
            b = k
            ob = 1 - k

            @pl.when(h + 1 < HIST)
            def _():
                @pl.when(h >= 1)
                def _():
                    wait_w(h - 1, ob)

                start_g(h + 1, ob)

            wait_g(h, b)
            start_w(h, b)
        return carry

    lax.fori_loop(0, HIST // 2, body, 0)
    wait_w(HIST - 2, 0)
    wait_w(HIST - 1, 1)


def kernel(x, table):
    xt = jnp.swapaxes(x.astype(jnp.int32), 0, 1)
    out = _sc_gather(table, xt)
    return jnp.swapaxes(out, 0, 1)


# speedup vs baseline: 10.6993x; 1.0244x over previous
import functools

import jax
import jax.numpy as jnp
from jax import lax
from jax.experimental import pallas as pl
from jax.experimental.pallas import tpu as pltpu
from jax.experimental.pallas import tpu_sc as plsc

NUM_EMBEDDINGS = 100000
DIM = 128
BATCH = 4096
HIST = 50

_info = plsc.get_sparse_core_info()
NC, NS = _info.num_cores, _info.num_subcores
NW = NC * NS
ROWS_PER_W = BATCH // NW


@functools.partial(
    pl.kernel,
    mesh=plsc.VectorSubcoreMesh(core_axis_name="c", subcore_axis_name="s"),
    out_type=jax.ShapeDtypeStruct((HIST, BATCH, DIM), jnp.float32),
    scratch_types=[
        pltpu.VMEM((HIST, ROWS_PER_W), jnp.int32),
        pltpu.VMEM((ROWS_PER_W, DIM), jnp.float32),
        pltpu.VMEM((ROWS_PER_W, DIM), jnp.float32),
        pltpu.VMEM((ROWS_PER_W, DIM), jnp.float32),
        pltpu.VMEM((ROWS_PER_W, DIM), jnp.float32),
        pltpu.SemaphoreType.DMA,
        pltpu.SemaphoreType.DMA,
        pltpu.SemaphoreType.DMA,
        pltpu.SemaphoreType.DMA,
        pltpu.SemaphoreType.DMA,
        pltpu.SemaphoreType.DMA,
        pltpu.SemaphoreType.DMA,
        pltpu.SemaphoreType.DMA,
    ],
)
def _sc_gather(
    tab_hbm, idx_hbm, out_hbm, idx_v,
    r0, r1, r2, r3, g0, g1, g2, g3, w0, w1, w2, w3,
):
    wid = lax.axis_index("s") * NC + lax.axis_index("c")
    base = wid * ROWS_PER_W
    rows = (r0, r1, r2, r3)
    gsem = (g0, g1, g2, g3)
    wsem = (w0, w1, w2, w3)
    pltpu.sync_copy(idx_hbm.at[:, pl.ds(base, ROWS_PER_W)], idx_v)

    def start_g(h, b):
        pltpu.async_copy(tab_hbm.at[idx_v.at[h]], rows[b], gsem[b])

    def wait_g(h, b):
        pltpu.make_async_copy(tab_hbm.at[idx_v.at[h]], rows[b], gsem[b]).wait()

    def start_w(h, b):
        pltpu.async_copy(rows[b], out_hbm.at[h, pl.ds(base, ROWS_PER_W)], wsem[b])

    def wait_w(h, b):
        pltpu.make_async_copy(
            rows[b], out_hbm.at[h, pl.ds(base, ROWS_PER_W)], wsem[b]
        ).wait()

    start_g(0, 0)
    start_g(1, 1)

    def body(o, carry):
        for k in range(4):
            h = 4 * o + k
            b = k
            nb = (k + 2) % 4

            @pl.when(h >= 2)
            def _():
                wait_w(h - 2, nb)

            start_g(h + 2, nb)
            wait_g(h, b)
            start_w(h, b)
        return carry

    lax.fori_loop(0, (HIST - 2) // 4, body, 0)
    for h in (HIST - 2, HIST - 1):
        b = h % 4
        wait_g(h, b)
        start_w(h, b)
    for h in range(HIST - 4, HIST):
        wait_w(h, h % 4)


def kernel(x, table):
    xt = jnp.swapaxes(x.astype(jnp.int32), 0, 1)
    out = _sc_gather(table, xt)
    return jnp.swapaxes(out, 0, 1)


# speedup vs baseline: 10.7285x; 1.0027x over previous
import functools

import jax
import jax.numpy as jnp
from jax import lax
from jax.experimental import pallas as pl
from jax.experimental.pallas import tpu as pltpu
from jax.experimental.pallas import tpu_sc as plsc

NUM_EMBEDDINGS = 100000
DIM = 128
BATCH = 4096
HIST = 50

_info = plsc.get_sparse_core_info()
NC, NS = _info.num_cores, _info.num_subcores
NW = NC * NS
ROWS_PER_W = BATCH // NW


@functools.partial(
    pl.kernel,
    mesh=plsc.VectorSubcoreMesh(core_axis_name="c", subcore_axis_name="s"),
    out_type=jax.ShapeDtypeStruct((HIST, BATCH, DIM), jnp.float32),
    scratch_types=[
        pltpu.VMEM((HIST, ROWS_PER_W), jnp.int32),
        pltpu.VMEM((ROWS_PER_W, DIM), jnp.float32),
        pltpu.VMEM((ROWS_PER_W, DIM), jnp.float32),
        pltpu.VMEM((ROWS_PER_W, DIM), jnp.float32),
        pltpu.VMEM((ROWS_PER_W, DIM), jnp.float32),
        pltpu.SemaphoreType.DMA,
        pltpu.SemaphoreType.DMA,
        pltpu.SemaphoreType.DMA,
        pltpu.SemaphoreType.DMA,
        pltpu.SemaphoreType.DMA,
        pltpu.SemaphoreType.DMA,
        pltpu.SemaphoreType.DMA,
        pltpu.SemaphoreType.DMA,
        pltpu.SemaphoreType.DMA,
    ],
)
def _sc_gather(
    tab_hbm, idx_hbm, out_hbm, idx_v,
    r0, r1, r2, r3, g0, g1, g2, g3, w0, w1, w2, w3, isem,
):
    wid = lax.axis_index("s") * NC + lax.axis_index("c")
    base = wid * ROWS_PER_W
    rows = (r0, r1, r2, r3)
    gsem = (g0, g1, g2, g3)
    wsem = (w0, w1, w2, w3)
    pltpu.sync_copy(idx_hbm.at[pl.ds(0, 8), pl.ds(base, ROWS_PER_W)],
                    idx_v.at[pl.ds(0, 8)])
    pltpu.async_copy(idx_hbm.at[pl.ds(8, HIST - 8), pl.ds(base, ROWS_PER_W)],
                     idx_v.at[pl.ds(8, HIST - 8)], isem)

    def start_g(h, b):
        pltpu.async_copy(tab_hbm.at[idx_v.at[h]], rows[b], gsem[b])

    def wait_g(h, b):
        pltpu.make_async_copy(tab_hbm.at[idx_v.at[h]], rows[b], gsem[b]).wait()

    def start_w(h, b):
        pltpu.async_copy(rows[b], out_hbm.at[h, pl.ds(base, ROWS_PER_W)], wsem[b])

    def wait_w(h, b):
        pltpu.make_async_copy(
            rows[b], out_hbm.at[h, pl.ds(base, ROWS_PER_W)], wsem[b]
        ).wait()

    start_g(0, 0)
    start_g(1, 1)
    pltpu.make_async_copy(
        idx_hbm.at[pl.ds(8, HIST - 8), pl.ds(base, ROWS_PER_W)],
        idx_v.at[pl.ds(8, HIST - 8)], isem,
    ).wait()

    def body(o, carry):
        for k in range(4):
            h = 4 * o + k
            b = k
            nb = (k + 2) % 4

            @pl.when(h >= 2)
            def _():
                wait_w(h - 2, nb)

            start_g(h + 2, nb)
            wait_g(h, b)
            start_w(h, b)
        return carry

    lax.fori_loop(0, (HIST - 2) // 4, body, 0)
    for h in (HIST - 2, HIST - 1):
        b = h % 4
        wait_g(h, b)
        start_w(h, b)
    for h in range(HIST - 4, HIST):
        wait_w(h, h % 4)


def kernel(x, table):
    xt = jnp.swapaxes(x.astype(jnp.int32), 0, 1)
    out = _sc_gather(table, xt)
    return jnp.swapaxes(out, 0, 1)


# speedup vs baseline: 10.8613x; 1.0124x over previous
import functools

import jax
import jax.numpy as jnp
from jax import lax
from jax.experimental import pallas as pl
from jax.experimental.pallas import tpu as pltpu
from jax.experimental.pallas import tpu_sc as plsc

NUM_EMBEDDINGS = 100000
DIM = 128
BATCH = 4096
HIST = 50

_info = plsc.get_sparse_core_info()
NC, NS = _info.num_cores, _info.num_subcores
NW = NC * NS
ROWS_PER_W = BATCH // NW
NPAIR = HIST // 2


@functools.partial(
    pl.kernel,
    mesh=plsc.VectorSubcoreMesh(core_axis_name="c", subcore_axis_name="s"),
    out_type=jax.ShapeDtypeStruct((HIST, BATCH, DIM), jnp.float32),
    scratch_types=[
        pltpu.VMEM((HIST, ROWS_PER_W), jnp.int32),
        pltpu.VMEM((2, ROWS_PER_W, DIM), jnp.float32),
        pltpu.VMEM((2, ROWS_PER_W, DIM), jnp.float32),
        pltpu.VMEM((2, ROWS_PER_W, DIM), jnp.float32),
        pltpu.SemaphoreType.DMA,
        pltpu.SemaphoreType.DMA,
        pltpu.SemaphoreType.DMA,
        pltpu.SemaphoreType.DMA,
        pltpu.SemaphoreType.DMA,
        pltpu.SemaphoreType.DMA,
        pltpu.SemaphoreType.DMA,
    ],
)
def _sc_gather(
    tab_hbm, idx_hbm, out_hbm, idx_v,
    r0, r1, r2, g0, g1, g2, w0, w1, w2, isem,
):
    wid = lax.axis_index("s") * NC + lax.axis_index("c")
    base = wid * ROWS_PER_W
    rows = (r0, r1, r2)
    gsem = (g0, g1, g2)
    wsem = (w0, w1, w2)
    pltpu.sync_copy(idx_hbm.at[pl.ds(0, 8), pl.ds(base, ROWS_PER_W)],
                    idx_v.at[pl.ds(0, 8)])
    pltpu.async_copy(idx_hbm.at[pl.ds(8, HIST - 8), pl.ds(base, ROWS_PER_W)],
                     idx_v.at[pl.ds(8, HIST - 8)], isem)

    def start_g(p, b):
        for j in (0, 1):
            pltpu.async_copy(
                tab_hbm.at[idx_v.at[2 * p + j]], rows[b].at[j], gsem[b]
            )

    def wait_g(p, b):
        for j in (0, 1):
            pltpu.make_async_copy(
                tab_hbm.at[idx_v.at[2 * p + j]], rows[b].at[j], gsem[b]
            ).wait()

    def start_w(p, b):
        pltpu.async_copy(
            rows[b], out_hbm.at[pl.ds(2 * p, 2), pl.ds(base, ROWS_PER_W)], wsem[b]
        )

    def wait_w(p, b):
        pltpu.make_async_copy(
            rows[b], out_hbm.at[pl.ds(2 * p, 2), pl.ds(base, ROWS_PER_W)], wsem[b]
        ).wait()

    start_g(0, 0)
    pltpu.make_async_copy(
        idx_hbm.at[pl.ds(8, HIST - 8), pl.ds(base, ROWS_PER_W)],
        idx_v.at[pl.ds(8, HIST - 8)], isem,
    ).wait()

    def body(o, carry):
        for k in range(3):
            p = 3 * o + k
            b = k
            nb = (k + 1) % 3

            @pl.when(p >= 2)
            def _():
                wait_w(p - 2, nb)

            start_g(p + 1, nb)
            wait_g(p, b)
            start_w(p, b)
        return carry

    lax.fori_loop(0, (NPAIR - 1) // 3, body, 0)
    p = NPAIR - 1
    wait_g(p, 0)
    start_w(p, 0)
    for q in (NPAIR - 3, NPAIR - 2, NPAIR - 1):
        wait_w(q, q % 3)


def kernel(x, table):
    xt = jnp.swapaxes(x.astype(jnp.int32), 0, 1)
    out = _sc_gather(table, xt)
    return jnp.swapaxes(out, 0, 1)
